# baseline (speedup 1.0000x reference)
import jax
import jax.numpy as jnp
from jax import lax
from jax.experimental import pallas as pl
from jax.experimental.pallas import tpu as pltpu
from jax.experimental.pallas import tpu_sc as plsc

DIM = 64
NF = 50
B = 16384
NC = 2
NS = 16
NW = NC * NS
BPW = B // NW
CHUNK = 128
NCHUNK = BPW // CHUNK
RING = 8


def _lanesum(x):
    idx = jnp.arange(16, dtype=jnp.int32)
    dnums = lax.GatherDimensionNumbers(
        offset_dims=(), collapsed_slice_dims=(0,), start_index_map=(0,))
    for sh in (8, 4, 2, 1):
        perm = jnp.bitwise_xor(idx, sh)
        x = x + lax.gather(x, perm[:, None], dimension_numbers=dnums,
                           slice_sizes=(1,),
                           mode=lax.GatherScatterMode.PROMISE_IN_BOUNDS)
    return x


def _rsqrt16(x):
    i = plsc.bitcast(x, jnp.int32)
    i = jnp.int32(0x5F3759DF) - jnp.right_shift(i, 1)
    y = plsc.bitcast(i, jnp.float32)
    half = x * jnp.float32(0.5)
    for _ in range(3):
        y = y * (jnp.float32(1.5) - half * y * y)
    return y


def _b16(v):
    return jnp.full((16,), v, jnp.int32)


def _body(idx_hbm, e_hbm, r_hbm, wk_hbm, out_hbm,
          idx_v, hr, rr, tr, wkv, score_v, sem):
    wid = lax.axis_index("s") * NC + lax.axis_index("c")
    base_row = wid * NCHUNK
    lanes = jnp.arange(16, dtype=jnp.int32)

    pltpu.sync_copy(idx_hbm.at[:, pl.ds(base_row, NCHUNK)], idx_v)
    pltpu.sync_copy(wk_hbm, wkv)

    s012 = jnp.minimum(lanes, 2)

    def idx3(i):
        g = plsc.load_gather(
            idx_v, [s012, _b16(lax.div(i, CHUNK)), _b16(lax.rem(i, CHUNK))])
        return g[0], g[1], g[2]

    def issue(j):
        ih, ir_, it = idx3(j)
        slot = jnp.bitwise_and(j, RING - 1) * 8
        ds = pl.ds(pl.multiple_of(slot, 8), 8)
        pltpu.async_copy(
            e_hbm.at[pl.ds(pl.multiple_of(
                lax.shift_left(lax.shift_right_logical(ih, 3), 3), 8), 8)],
            hr.at[ds], sem)
        pltpu.async_copy(
            r_hbm.at[pl.ds(pl.multiple_of(
                lax.shift_left(lax.shift_right_logical(ir_, 3), 3), 8), 8)],
            rr.at[ds], sem)
        pltpu.async_copy(
            e_hbm.at[pl.ds(pl.multiple_of(
                lax.shift_left(lax.shift_right_logical(it, 3), 3), 8), 8)],
            tr.at[ds], sem)

    def drain3():
        for ref in (hr, rr, tr):
            pltpu.make_async_copy(
                e_hbm.at[pl.ds(0, 8)], ref.at[pl.ds(0, 8)], sem).wait()

    for j in range(RING):
        issue(j)

    lane0 = lanes == 0

    def triple(i, carry):
        drain3()
        ih, ir_, it = idx3(i)
        slot = jnp.bitwise_and(i, RING - 1) * 8

        def rows(sub, ref):
            base = slot + sub
            return [plsc.load_gather(ref, [_b16(base), _b16(16 * k) + lanes])
                    for k in range(4)]

        h = rows(jnp.bitwise_and(ih, 7), hr)
        r = rows(jnp.bitwise_and(ir_, 7), rr)
        t = rows(jnp.bitwise_and(it, 7), tr)

        def nxt(_):
            issue(i + RING)
            return 0

        lax.cond(i + RING < BPW, nxt, lambda _: 0, 0)

        def inv_norm(x):
            ssq = x[0] * x[0] + x[1] * x[1] + x[2] * x[2] + x[3] * x[3]
            s = _lanesum(ssq)
            return _rsqrt16(jnp.maximum(s, jnp.float32(1e-24)))

        ihn, irn, itn = inv_norm(h), inv_norm(r), inv_norm(t)
        h = [x * ihn for x in h]
        r = [x * irn for x in r]
        t = [x * itn for x in t]

        acc = [jnp.zeros((16,), jnp.float32) for _ in range(4)]
        for f in range(NF):
            ka = wkv[f, pl.ds(DIM, 16)]
            kb = wkv[f, pl.ds(DIM + 16, 16)]
            kc = wkv[f, pl.ds(DIM + 32, 16)]
            for k in range(4):
                z = h[k] * ka + r[k] * kb + t[k] * kc
                z = jnp.maximum(z, jnp.float32(0.0))
                acc[k] = acc[k] + z * wkv[f, pl.ds(16 * k, 16)]
        tot = -_lanesum(acc[0] + acc[1] + acc[2] + acc[3])
        plsc.store_scatter(score_v, [_b16(i)], tot, mask=lane0)
        return carry

    lax.fori_loop(0, BPW, triple, 0)
    pltpu.sync_copy(score_v, out_hbm.at[pl.ds(wid * BPW, BPW)])


TCOLS = 2048


def _transpose_body(inv, outv):
    outv[...] = inv[...].T


def _to_rows(table_t):
    n = table_t.shape[1]
    return pl.pallas_call(
        _transpose_body,
        grid=((n + TCOLS - 1) // TCOLS,),
        in_specs=[pl.BlockSpec((DIM, TCOLS), lambda j: (0, j))],
        out_specs=pl.BlockSpec((TCOLS, DIM), lambda j: (j, 0)),
        out_shape=jax.ShapeDtypeStruct((n, DIM), jnp.float32),
    )(table_t)


def kernel(T, E_table, R_table, kernel, fc_W):
    idx = T.T.reshape(3, B // CHUNK, CHUNK).astype(jnp.int32)
    k3 = kernel[:, 0, 0, :]
    kbt = jnp.repeat(k3, 16, axis=1).astype(jnp.float32)
    W = fc_W.reshape(NF, DIM)
    wk = jnp.concatenate(
        [W, kbt, jnp.zeros((NF, 128 - DIM - 48), jnp.float32)], axis=1)

    mesh = plsc.VectorSubcoreMesh(core_axis_name="c", subcore_axis_name="s")
    run = pl.kernel(
        _body,
        out_type=jax.ShapeDtypeStruct((B,), jnp.float32),
        mesh=mesh,
        compiler_params=pltpu.CompilerParams(needs_layout_passes=False,
                                             use_tc_tiling_on_sc=True),
        scratch_types=[
            pltpu.VMEM((3, NCHUNK, CHUNK), jnp.int32),
            pltpu.VMEM((RING * 8, DIM), jnp.float32),
            pltpu.VMEM((RING * 8, DIM), jnp.float32),
            pltpu.VMEM((RING * 8, DIM), jnp.float32),
            pltpu.VMEM((NF, 128), jnp.float32),
            pltpu.VMEM((BPW,), jnp.float32),
            pltpu.SemaphoreType.DMA,
        ],
    )
    return run(idx, _to_rows(E_table.T), _to_rows(R_table.T), wk)


# speedup vs baseline: 1.2792x; 1.2792x over previous
import jax
import jax.numpy as jnp
from jax import lax
from jax.experimental import pallas as pl
from jax.experimental.pallas import tpu as pltpu
from jax.experimental.pallas import tpu_sc as plsc

DIM = 64
NF = 50
B = 16384
NC = 2
NS = 16
NW = NC * NS
BPW = B // NW
CHUNK = 128
NCHUNK = BPW // CHUNK
RING = 16


def _lanesum(x):
    idx = jnp.arange(16, dtype=jnp.int32)
    dnums = lax.GatherDimensionNumbers(
        offset_dims=(), collapsed_slice_dims=(0,), start_index_map=(0,))
    for sh in (8, 4, 2, 1):
        perm = jnp.bitwise_xor(idx, sh)
        x = x + lax.gather(x, perm[:, None], dimension_numbers=dnums,
                           slice_sizes=(1,),
                           mode=lax.GatherScatterMode.PROMISE_IN_BOUNDS)
    return x


def _rsqrt16(x):
    i = plsc.bitcast(x, jnp.int32)
    i = jnp.int32(0x5F3759DF) - jnp.right_shift(i, 1)
    y = plsc.bitcast(i, jnp.float32)
    half = x * jnp.float32(0.5)
    for _ in range(3):
        y = y * (jnp.float32(1.5) - half * y * y)
    return y


def _b16(v):
    return jnp.full((16,), v, jnp.int32)


def _body(idx_hbm, e_hbm, r_hbm, wk_hbm, out_hbm,
          idx_v, hr, rr, tr, wkv, score_v, sem):
    wid = lax.axis_index("s") * NC + lax.axis_index("c")
    base_row = wid * NCHUNK
    lanes = jnp.arange(16, dtype=jnp.int32)

    pltpu.sync_copy(idx_hbm.at[:, pl.ds(base_row, NCHUNK)], idx_v)
    pltpu.sync_copy(wk_hbm, wkv)

    s012 = jnp.minimum(lanes, 2)

    def idx3(i):
        g = plsc.load_gather(
            idx_v, [s012, _b16(lax.div(i, CHUNK)), _b16(lax.rem(i, CHUNK))])
        return g[0], g[1], g[2]

    def issue(j):
        ih, ir_, it = idx3(j)
        slot = jnp.bitwise_and(j, RING - 1) * 8
        ds = pl.ds(pl.multiple_of(slot, 8), 8)
        pltpu.async_copy(
            e_hbm.at[pl.ds(pl.multiple_of(
                lax.shift_left(lax.shift_right_logical(ih, 3), 3), 8), 8)],
            hr.at[ds], sem)
        pltpu.async_copy(
            r_hbm.at[pl.ds(pl.multiple_of(
                lax.shift_left(lax.shift_right_logical(ir_, 3), 3), 8), 8)],
            rr.at[ds], sem)
        pltpu.async_copy(
            e_hbm.at[pl.ds(pl.multiple_of(
                lax.shift_left(lax.shift_right_logical(it, 3), 3), 8), 8)],
            tr.at[ds], sem)

    def drain3():
        for ref in (hr, rr, tr):
            pltpu.make_async_copy(
                e_hbm.at[pl.ds(0, 8)], ref.at[pl.ds(0, 8)], sem).wait()

    for j in range(RING):
        issue(j)

    lane0 = lanes == 0

    def triple(i, carry):
        drain3()
        ih, ir_, it = idx3(i)
        slot = jnp.bitwise_and(i, RING - 1) * 8

        def rows(sub, ref):
            base = slot + sub
            return [plsc.load_gather(ref, [_b16(base), _b16(16 * k) + lanes])
                    for k in range(4)]

        h = rows(jnp.bitwise_and(ih, 7), hr)
        r = rows(jnp.bitwise_and(ir_, 7), rr)
        t = rows(jnp.bitwise_and(it, 7), tr)

        def nxt(_):
            issue(i + RING)
            return 0

        lax.cond(i + RING < BPW, nxt, lambda _: 0, 0)

        def inv_norm(x):
            ssq = x[0] * x[0] + x[1] * x[1] + x[2] * x[2] + x[3] * x[3]
            s = _lanesum(ssq)
            return _rsqrt16(jnp.maximum(s, jnp.float32(1e-24)))

        ihn, irn, itn = inv_norm(h), inv_norm(r), inv_norm(t)
        h = [x * ihn for x in h]
        r = [x * irn for x in r]
        t = [x * itn for x in t]

        acc = [jnp.zeros((16,), jnp.float32) for _ in range(4)]
        for f in range(NF):
            ka = wkv[f, pl.ds(DIM, 16)]
            kb = wkv[f, pl.ds(DIM + 16, 16)]
            kc = wkv[f, pl.ds(DIM + 32, 16)]
            for k in range(4):
                z = h[k] * ka + r[k] * kb + t[k] * kc
                z = jnp.maximum(z, jnp.float32(0.0))
                acc[k] = acc[k] + z * wkv[f, pl.ds(16 * k, 16)]
        tot = -_lanesum(acc[0] + acc[1] + acc[2] + acc[3])
        plsc.store_scatter(score_v, [_b16(i)], tot, mask=lane0)
        return carry

    lax.fori_loop(0, BPW, triple, 0)
    pltpu.sync_copy(score_v, out_hbm.at[pl.ds(wid * BPW, BPW)])


def kernel(T, E_table, R_table, kernel, fc_W):
    idx = T.T.reshape(3, B // CHUNK, CHUNK).astype(jnp.int32)
    k3 = kernel[:, 0, 0, :]
    kbt = jnp.repeat(k3, 16, axis=1).astype(jnp.float32)
    W = fc_W.reshape(NF, DIM)
    wk = jnp.concatenate(
        [W, kbt, jnp.zeros((NF, 128 - DIM - 48), jnp.float32)], axis=1)

    mesh = plsc.VectorSubcoreMesh(core_axis_name="c", subcore_axis_name="s")
    run = pl.kernel(
        _body,
        out_type=jax.ShapeDtypeStruct((B,), jnp.float32),
        mesh=mesh,
        compiler_params=pltpu.CompilerParams(needs_layout_passes=False,
                                             use_tc_tiling_on_sc=True),
        scratch_types=[
            pltpu.VMEM((3, NCHUNK, CHUNK), jnp.int32),
            pltpu.VMEM((RING * 8, DIM), jnp.float32),
            pltpu.VMEM((RING * 8, DIM), jnp.float32),
            pltpu.VMEM((RING * 8, DIM), jnp.float32),
            pltpu.VMEM((NF, 128), jnp.float32),
            pltpu.VMEM((BPW,), jnp.float32),
            pltpu.SemaphoreType.DMA,
        ],
    )
    return run(idx, E_table, R_table, wk)


# speedup vs baseline: 1.2817x; 1.0020x over previous
import jax
import jax.numpy as jnp
from jax import lax
from jax.experimental import pallas as pl
from jax.experimental.pallas import tpu as pltpu
from jax.experimental.pallas import tpu_sc as plsc

DIM = 64
NF = 50
B = 16384
NC = 2
NS = 16
NW = NC * NS
BPW = B // NW
CHUNK = 128
NCHUNK = BPW // CHUNK
RING = 8


def _lanesum(x):
    idx = jnp.arange(16, dtype=jnp.int32)
    dnums = lax.GatherDimensionNumbers(
        offset_dims=(), collapsed_slice_dims=(0,), start_index_map=(0,))
    for sh in (8, 4, 2, 1):
        perm = jnp.bitwise_xor(idx, sh)
        x = x + lax.gather(x, perm[:, None], dimension_numbers=dnums,
                           slice_sizes=(1,),
                           mode=lax.GatherScatterMode.PROMISE_IN_BOUNDS)
    return x


def _rsqrt16(x):
    i = plsc.bitcast(x, jnp.int32)
    i = jnp.int32(0x5F3759DF) - jnp.right_shift(i, 1)
    y = plsc.bitcast(i, jnp.float32)
    half = x * jnp.float32(0.5)
    for _ in range(3):
        y = y * (jnp.float32(1.5) - half * y * y)
    return y


def _b16(v):
    return jnp.full((16,), v, jnp.int32)


def _body(idx_hbm, e_hbm, r_hbm, wk_hbm, out_hbm,
          idx_v, hr, rr, tr, wkv, score_v, sem):
    wid = lax.axis_index("s") * NC + lax.axis_index("c")
    base_row = wid * NCHUNK
    lanes = jnp.arange(16, dtype=jnp.int32)

    pltpu.sync_copy(idx_hbm.at[:, pl.ds(base_row, NCHUNK)], idx_v)
    pltpu.sync_copy(wk_hbm, wkv)

    s012 = jnp.minimum(lanes, 2)

    def idx3(i):
        g = plsc.load_gather(
            idx_v, [s012, _b16(lax.div(i, CHUNK)), _b16(lax.rem(i, CHUNK))])
        return g[0], g[1], g[2]

    def issue(j):
        ih, ir_, it = idx3(j)
        slot = jnp.bitwise_and(j, RING - 1) * 8
        ds = pl.ds(pl.multiple_of(slot, 8), 8)
        pltpu.async_copy(
            e_hbm.at[pl.ds(pl.multiple_of(
                lax.shift_left(lax.shift_right_logical(ih, 3), 3), 8), 8)],
            hr.at[ds], sem)
        pltpu.async_copy(
            r_hbm.at[pl.ds(pl.multiple_of(
                lax.shift_left(lax.shift_right_logical(ir_, 3), 3), 8), 8)],
            rr.at[ds], sem)
        pltpu.async_copy(
            e_hbm.at[pl.ds(pl.multiple_of(
                lax.shift_left(lax.shift_right_logical(it, 3), 3), 8), 8)],
            tr.at[ds], sem)

    def drain3():
        for ref in (hr, rr, tr):
            pltpu.make_async_copy(
                e_hbm.at[pl.ds(0, 8)], ref.at[pl.ds(0, 8)], sem).wait()

    for j in range(RING):
        issue(j)

    lane0 = lanes == 0

    def triple(i, carry):
        drain3()
        ih, ir_, it = idx3(i)
        slot = jnp.bitwise_and(i, RING - 1) * 8

        def rows(sub, ref):
            base = slot + sub
            return [plsc.load_gather(ref, [_b16(base), _b16(16 * k) + lanes])
                    for k in range(4)]

        h = rows(jnp.bitwise_and(ih, 7), hr)
        r = rows(jnp.bitwise_and(ir_, 7), rr)
        t = rows(jnp.bitwise_and(it, 7), tr)

        def nxt(_):
            issue(i + RING)
            return 0

        lax.cond(i + RING < BPW, nxt, lambda _: 0, 0)

        def inv_norm(x):
            ssq = x[0] * x[0] + x[1] * x[1] + x[2] * x[2] + x[3] * x[3]
            s = _lanesum(ssq)
            return _rsqrt16(jnp.maximum(s, jnp.float32(1e-24)))

        ihn, irn, itn = inv_norm(h), inv_norm(r), inv_norm(t)
        h = [x * ihn for x in h]
        r = [x * irn for x in r]
        t = [x * itn for x in t]

        acc = [jnp.zeros((16,), jnp.float32) for _ in range(4)]
        for f in range(NF):
            ka = wkv[f, pl.ds(DIM, 16)]
            kb = wkv[f, pl.ds(DIM + 16, 16)]
            kc = wkv[f, pl.ds(DIM + 32, 16)]
            for k in range(4):
                z = h[k] * ka + r[k] * kb + t[k] * kc
                z = jnp.maximum(z, jnp.float32(0.0))
                acc[k] = acc[k] + z * wkv[f, pl.ds(16 * k, 16)]
        tot = -_lanesum(acc[0] + acc[1] + acc[2] + acc[3])
        plsc.store_scatter(score_v, [_b16(i)], tot, mask=lane0)
        return carry

    lax.fori_loop(0, BPW, triple, 0)
    pltpu.sync_copy(score_v, out_hbm.at[pl.ds(wid * BPW, BPW)])


def kernel(T, E_table, R_table, kernel, fc_W):
    idx = T.T.reshape(3, B // CHUNK, CHUNK).astype(jnp.int32)
    k3 = kernel[:, 0, 0, :]
    kbt = jnp.repeat(k3, 16, axis=1).astype(jnp.float32)
    W = fc_W.reshape(NF, DIM)
    wk = jnp.concatenate(
        [W, kbt, jnp.zeros((NF, 128 - DIM - 48), jnp.float32)], axis=1)

    mesh = plsc.VectorSubcoreMesh(core_axis_name="c", subcore_axis_name="s")
    run = pl.kernel(
        _body,
        out_type=jax.ShapeDtypeStruct((B,), jnp.float32),
        mesh=mesh,
        compiler_params=pltpu.CompilerParams(needs_layout_passes=False,
                                             use_tc_tiling_on_sc=True),
        scratch_types=[
            pltpu.VMEM((3, NCHUNK, CHUNK), jnp.int32),
            pltpu.VMEM((RING * 8, DIM), jnp.float32),
            pltpu.VMEM((RING * 8, DIM), jnp.float32),
            pltpu.VMEM((RING * 8, DIM), jnp.float32),
            pltpu.VMEM((NF, 128), jnp.float32),
            pltpu.VMEM((BPW,), jnp.float32),
            pltpu.SemaphoreType.DMA,
        ],
    )
    return run(idx, E_table, R_table, wk)
